# Initial kernel scaffold; baseline (speedup 1.0000x reference)
import jax
import jax.numpy as jnp
from jax.experimental import pallas as pl


def kernel(features, edge_index, edge_attr, time_embedding, W1, b1, W2, b2, F1, fb1, F2, fb2):
    raise NotImplementedError("write your pallas kernel here")


# speedup vs baseline: 3.5397x; 3.5397x over previous
import jax
import jax.numpy as jnp
from jax import lax
from jax.experimental import pallas as pl
from jax.experimental.pallas import tpu as pltpu
from jax.experimental.pallas import tpu_sc as plsc

N = 10000
E = 320000
D = 128

NP = 10240
EP = 320512

NC = 2
NS = 16
NW = NC * NS

SUP = 1024
NCHUNK = EP // SUP
TRIPS = -(-NCHUNK // NW)
G = 80
NG = N // G
GTRIPS = -(-NG // NS)

_f32 = jnp.float32


def _sigmoid(x):
    return jax.nn.sigmoid(x)


def _softsign(x):
    return x / (1.0 + jnp.abs(x))


def _pre_body(f_ref, w1a_ref, w1b_ref, p_ref, q_ref):
    f = f_ref[...]
    p_ref[...] = jnp.dot(f, w1a_ref[...], preferred_element_type=_f32)
    q_ref[...] = jnp.dot(f, w1b_ref[...], preferred_element_type=_f32)


def _msg_body(zs_ref, zt_ref, b1_ref, w2_ref, b2_ref, o_ref):
    z = zs_ref[...] + zt_ref[...] + b1_ref[...]
    h = _sigmoid(z)
    m = jnp.dot(h, w2_ref[...], preferred_element_type=_f32) + b2_ref[...]
    o_ref[...] = _softsign(m)


def _fin_body(f_ref, p0_ref, p1_ref, t_ref, f1a_ref, f1b_ref, f1c_ref,
              fb1_ref, f2_ref, fb2_ref, o_ref):
    gf = _sigmoid(f_ref[...])
    ga = _sigmoid(p0_ref[...] + p1_ref[...])
    gt = _sigmoid(t_ref[...])
    g = _sigmoid(jnp.dot(gf, f1a_ref[...], preferred_element_type=_f32)
                 + jnp.dot(ga, f1b_ref[...], preferred_element_type=_f32)
                 + jnp.dot(gt, f1c_ref[...], preferred_element_type=_f32)
                 + fb1_ref[...])
    y = jnp.dot(g, f2_ref[...], preferred_element_type=_f32) + fb2_ref[...]
    o_ref[...] = _softsign(y)


def _row_spec(rows):
    return pl.BlockSpec((rows, D), lambda i: (i, 0))


def _rep_spec(shape):
    return pl.BlockSpec(shape, lambda i: (0,) * len(shape))


def _gather_body(rows2_hbm, cols2_hbm, p_hbm, q_hbm, zs_hbm, zt_hbm,
                 idxr, idxc, bufp, bufq, semp, semq):
    w = lax.axis_index("s") * NC + lax.axis_index("c")

    def chunk(t, _):
        c = t * NW + w

        @pl.when(c < NCHUNK)
        def _():
            pltpu.sync_copy(rows2_hbm.at[pl.ds(c * 8, 8)], idxr)
            pltpu.sync_copy(cols2_hbm.at[pl.ds(c * 8, 8)], idxc)
            for i in range(4):
                cp = []
                for j in range(2):
                    cp.append(pltpu.async_copy(
                        p_hbm.at[idxr.at[2 * i + j]],
                        bufp.at[pl.ds(j * 128, 128)], semp))
                    cp.append(pltpu.async_copy(
                        q_hbm.at[idxc.at[2 * i + j]],
                        bufq.at[pl.ds(j * 128, 128)], semq))
                for d in cp:
                    d.wait()
                base = c * SUP + i * 256
                pltpu.sync_copy(bufp, zs_hbm.at[pl.ds(base, 256)])
                pltpu.sync_copy(bufq, zt_hbm.at[pl.ds(base, 256)])

        return ()

    lax.fori_loop(0, TRIPS, chunk, ())


def _scatter_body(rows2_hbm, msg_hbm, zeros_hbm, out_hbm, idx, buf, acc, sem):
    cid = lax.axis_index("c")
    sid = lax.axis_index("s")
    w = sid * NC + cid

    @pl.when(sid == 0)
    def _():
        pltpu.sync_copy(zeros_hbm, acc)

    plsc.subcore_barrier()

    def chunk(t, _):
        c = t * NW + w

        @pl.when(c < NCHUNK)
        def _():
            pltpu.sync_copy(rows2_hbm.at[pl.ds(c * 8, 8)], idx)
            for i in range(4):
                base = c * SUP + i * 256
                pltpu.sync_copy(msg_hbm.at[pl.ds(base, 256)], buf)
                for j in range(2):
                    pltpu.sync_copy(buf.at[pl.ds(j * 128, 128)],
                                    acc.at[idx.at[2 * i + j]], add=True)

        return ()

    lax.fori_loop(0, TRIPS, chunk, ())
    plsc.subcore_barrier()

    def wb(t, _):
        g = t * NS + sid

        @pl.when(g < NG)
        def _():
            pltpu.sync_copy(acc.at[pl.ds(g * G, G)],
                            out_hbm.at[cid, pl.ds(g * G, G)])

        return ()

    lax.fori_loop(0, GTRIPS, wb, ())


_SC_MESH = plsc.VectorSubcoreMesh(core_axis_name="c", subcore_axis_name="s",
                                  num_cores=NC, num_subcores=NS)

_gather = pl.kernel(
    _gather_body,
    out_type=[jax.ShapeDtypeStruct((EP, D), _f32),
              jax.ShapeDtypeStruct((EP, D), _f32)],
    mesh=_SC_MESH,
    scratch_types=[
        pltpu.VMEM((8, 128), jnp.int32),
        pltpu.VMEM((8, 128), jnp.int32),
        pltpu.VMEM((256, D), _f32),
        pltpu.VMEM((256, D), _f32),
        pltpu.SemaphoreType.DMA,
        pltpu.SemaphoreType.DMA,
    ],
    name="sc_edge_gather",
)

_scatter = pl.kernel(
    _scatter_body,
    out_type=jax.ShapeDtypeStruct((NC, N, D), _f32),
    mesh=_SC_MESH,
    scratch_types=[
        pltpu.VMEM((8, 128), jnp.int32),
        pltpu.VMEM((256, D), _f32),
        pltpu.VMEM_SHARED((NP, D), _f32),
        pltpu.SemaphoreType.DMA,
    ],
    name="sc_segment_sum",
)


def kernel(features, edge_index, edge_attr, time_embedding,
           W1, b1, W2, b2, F1, fb1, F2, fb2):
    del edge_attr
    rows = edge_index[0]
    cols = edge_index[1]
    pad_e = EP - E
    rows_p = jnp.concatenate([rows, jnp.full((pad_e,), N, jnp.int32)])
    cols_p = jnp.concatenate([cols, jnp.zeros((pad_e,), jnp.int32)])
    rows2 = rows_p.reshape(EP // 128, 128)
    cols2 = cols_p.reshape(EP // 128, 128)
    feats_p = jnp.concatenate([features, jnp.zeros((NP - N, D), _f32)])
    b1r = b1.reshape(1, D)
    b2r = b2.reshape(1, D)
    fb1r = fb1.reshape(1, D)
    fb2r = fb2.reshape(1, D)
    w1a = W1[:D]
    w1b = W1[D:]
    f1a = F1[:D]
    f1b = F1[D:2 * D]
    f1c = F1[2 * D:]

    p, q = pl.pallas_call(
        _pre_body,
        grid=(10,),
        in_specs=[_row_spec(1024), _rep_spec((D, D)), _rep_spec((D, D))],
        out_specs=[_row_spec(1024), _row_spec(1024)],
        out_shape=[jax.ShapeDtypeStruct((NP, D), _f32),
                   jax.ShapeDtypeStruct((NP, D), _f32)],
    )(feats_p, w1a, w1b)

    zs, zt = _gather(rows2, cols2, p, q)

    msg = pl.pallas_call(
        _msg_body,
        grid=(313,),
        in_specs=[_row_spec(1024), _row_spec(1024), _rep_spec((1, D)),
                  _rep_spec((D, D)), _rep_spec((1, D))],
        out_specs=_row_spec(1024),
        out_shape=jax.ShapeDtypeStruct((EP, D), _f32),
    )(zs, zt, b1r, W2, b2r)

    partials = _scatter(rows2, msg, jnp.zeros((NP, D), _f32))

    out = pl.pallas_call(
        _fin_body,
        grid=(10,),
        in_specs=[_row_spec(1000), _row_spec(1000), _row_spec(1000),
                  _row_spec(1000), _rep_spec((D, D)), _rep_spec((D, D)),
                  _rep_spec((D, D)), _rep_spec((1, D)), _rep_spec((D, D)),
                  _rep_spec((1, D))],
        out_specs=_row_spec(1000),
        out_shape=jax.ShapeDtypeStruct((N, D), _f32),
    )(features, partials[0], partials[1], time_embedding,
      f1a, f1b, f1c, fb1r, F2, fb2r)

    return out
